# Initial kernel scaffold; baseline (speedup 1.0000x reference)
import jax
import jax.numpy as jnp
from jax.experimental import pallas as pl


def kernel(weeks, seasons, holidays_tab, w1, w2, w3, week_idx, day_idx, holiday_idx):
    raise NotImplementedError("write your pallas kernel here")


# speedup vs baseline: 3.4092x; 3.4092x over previous
import jax
import jax.numpy as jnp
from jax import lax
from jax.experimental import pallas as pl
from jax.experimental.pallas import tpu as pltpu
from jax.experimental.pallas import tpu_sc as plsc

B = 16384
D = 24
NC = 2
NS = 16
NW = NC * NS
BPW = B // NW
L = 16


def _sc_body(weeks_hbm, seasons_hbm, hol_hbm, w_hbm,
             wk_idx_hbm, dy_idx_hbm, hl_idx_hbm,
             out_hbm,
             weeks_v, seasons_v, hol_v, w_v,
             wk_v, dy_v, hl_v, out_v, sem):
    wid = lax.axis_index("s") * NC + lax.axis_index("c")
    base = wid * BPW

    pltpu.sync_copy(weeks_hbm, weeks_v)
    pltpu.sync_copy(seasons_hbm, seasons_v)
    pltpu.sync_copy(hol_hbm, hol_v)
    pltpu.sync_copy(w_hbm, w_v)
    pltpu.sync_copy(wk_idx_hbm.at[pl.ds(base, BPW)], wk_v)
    pltpu.sync_copy(dy_idx_hbm.at[pl.ds(base, BPW)], dy_v)
    pltpu.sync_copy(hl_idx_hbm.at[pl.ds(base, BPW)], hl_v)

    w1 = w_v[pl.ds(0, L)]
    w2 = w_v[pl.ds(L, L)]
    w3 = w_v[pl.ds(2 * L, L)]
    lane = lax.iota(jnp.int32, L)

    def group(g, carry):
        b0 = g * L
        wk = wk_v[pl.ds(b0, L)] * D
        dy = dy_v[pl.ds(b0, L)] * D
        hl = hl_v[pl.ds(b0, L)] * D
        rows = (b0 + lane) * D
        for d in range(D):
            a = plsc.load_gather(weeks_v, [wk + d])
            b = plsc.load_gather(seasons_v, [dy + d])
            c = plsc.load_gather(hol_v, [hl + d])
            val = w1 * a + w2 * b + w3 * c
            plsc.store_scatter(out_v, [rows + d], val)
        return carry

    lax.fori_loop(0, BPW // L, group, 0)

    pltpu.sync_copy(out_v, out_hbm.at[pl.ds(base * D, BPW * D)])


def kernel(weeks, seasons, holidays_tab, w1, w2, w3, week_idx, day_idx, holiday_idx):
    w = jnp.broadcast_to(jnp.stack([w1, w2, w3])[:, None], (3, L)).reshape(-1)
    mesh = plsc.VectorSubcoreMesh(core_axis_name="c", subcore_axis_name="s")
    f = pl.kernel(
        _sc_body,
        mesh=mesh,
        compiler_params=pltpu.CompilerParams(needs_layout_passes=False),
        out_type=jax.ShapeDtypeStruct((B * D,), jnp.float32),
        scratch_types=[
            pltpu.VMEM((53 * D,), jnp.float32),
            pltpu.VMEM((7 * D,), jnp.float32),
            pltpu.VMEM((2 * D,), jnp.float32),
            pltpu.VMEM((3 * L,), jnp.float32),
            pltpu.VMEM((BPW,), jnp.int32),
            pltpu.VMEM((BPW,), jnp.int32),
            pltpu.VMEM((BPW,), jnp.int32),
            pltpu.VMEM((BPW * D,), jnp.float32),
            pltpu.SemaphoreType.DMA,
        ],
    )
    out = f(weeks.reshape(-1), seasons.reshape(-1), holidays_tab.reshape(-1), w,
            week_idx, day_idx, holiday_idx)
    return out.reshape(B, D)
